# baseline (speedup 1.0000x reference)
import functools

import jax
import jax.numpy as jnp
from jax.experimental import pallas as pl
from jax.experimental.pallas import tpu as pltpu


def _row_mean_block_kernel(x_ref, o_ref, *, inv_hw):
    s = jnp.sum(x_ref[...], axis=-1, keepdims=True, dtype=jnp.float32)
    o_ref[...] = (s * inv_hw).astype(o_ref.dtype)


def _pick_row_tile(rows, hw, itemsize, target_bytes):
    if rows * hw * itemsize <= target_bytes:
        return rows
    tr = max(8, target_bytes // (max(hw, 128) * itemsize))
    tr = (tr // 8) * 8
    while tr > 8 and rows % tr != 0:
        tr -= 8
    if rows % tr != 0:
        tr = rows
    return tr


def kernel(x):
    N, C, H, W = x.shape
    rows, hw = N * C, H * W
    x2d = x.reshape(rows, hw)
    itemsize = x2d.dtype.itemsize

    tr = _pick_row_tile(rows, hw, itemsize, target_bytes=8 * 1024 * 1024)
    grid = pl.cdiv(rows, tr)

    out2d = pl.pallas_call(
        functools.partial(_row_mean_block_kernel, inv_hw=1.0 / hw),
        out_shape=jax.ShapeDtypeStruct((rows, 1), x.dtype),
        grid=(grid,),
        in_specs=[pl.BlockSpec((tr, hw), lambda i: (i, 0))],
        out_specs=pl.BlockSpec((tr, 1), lambda i: (i, 0)),
        compiler_params=pltpu.CompilerParams(
            dimension_semantics=("parallel",),
            vmem_limit_bytes=64 * 1024 * 1024,
        ),
        cost_estimate=pl.CostEstimate(
            flops=rows * hw,
            transcendentals=0,
            bytes_accessed=rows * hw * itemsize + rows * itemsize,
        ),
    )(x2d)

    return out2d.reshape(N, C, 1, 1)


# speedup vs baseline: 1.0759x; 1.0759x over previous
import functools

import jax
import jax.numpy as jnp
from jax.experimental import pallas as pl
from jax.experimental.pallas import tpu as pltpu


def _row_mean_block_kernel(x_ref, o_ref, *, inv_hw):
    s = jnp.sum(x_ref[...], axis=-1, dtype=jnp.float32)
    o_ref[...] = (s * inv_hw).astype(o_ref.dtype).reshape(o_ref.shape)


def _pick_row_tile(rows, hw, itemsize, target_bytes):
    if rows * hw * itemsize <= target_bytes:
        return rows
    tr = max(8, target_bytes // (max(hw, 128) * itemsize))
    tr = (tr // 8) * 8
    while tr > 8 and rows % tr != 0:
        tr -= 8
    if rows % tr != 0:
        tr = rows
    return tr


def kernel(x):
    N, C, H, W = x.shape
    rows, hw = N * C, H * W
    x2d = x.reshape(rows, hw)
    itemsize = x2d.dtype.itemsize

    tr = _pick_row_tile(rows, hw, itemsize, target_bytes=8 * 1024 * 1024)
    grid = pl.cdiv(rows, tr)

    out3d = pl.pallas_call(
        functools.partial(_row_mean_block_kernel, inv_hw=1.0 / hw),
        out_shape=jax.ShapeDtypeStruct((grid, 1, tr), x.dtype),
        grid=(grid,),
        in_specs=[pl.BlockSpec((tr, hw), lambda i: (i, 0))],
        out_specs=pl.BlockSpec((1, 1, tr), lambda i: (i, 0, 0)),
        compiler_params=pltpu.CompilerParams(
            dimension_semantics=("parallel",),
            vmem_limit_bytes=64 * 1024 * 1024,
        ),
        cost_estimate=pl.CostEstimate(
            flops=rows * hw,
            transcendentals=0,
            bytes_accessed=rows * hw * itemsize + rows * itemsize,
        ),
    )(x2d)

    return out3d.reshape(N, C, 1, 1)
